# baseline (speedup 1.0000x reference)
import functools

import jax
import jax.numpy as jnp
from jax import lax
from jax.experimental import pallas as pl
from jax.experimental.pallas import tpu as pltpu
from jax.experimental.pallas import tpu_sc as plsc

IMAGE_SIZE = 300
D_POS = 64
DG = 128
T = 100
B = 1024
ROWS = B * T
BBOX_DIM = 4 * D_POS
OUT_DIM = 512
ZROW = IMAGE_SIZE + 1

CHUNK_R = 128
NW = 32
R_PER_W = ROWS // NW
NCHUNK = R_PER_W // CHUNK_R


def _sc_gather_body(frac_hbm, obj_hbm, table_hbm, poe_hbm, frac_v, obj_v,
                    idx_v, rows_v, sem):
    wid = lax.axis_index("s") * 2 + lax.axis_index("c")
    r0 = wid * R_PER_W

    def chunk(ci, carry):
        off = r0 + ci * CHUNK_R
        for c in range(4):
            pltpu.sync_copy(frac_hbm.at[c, pl.ds(off, CHUNK_R)],
                            frac_v.at[c])
        pltpu.sync_copy(obj_hbm.at[pl.ds(off, CHUNK_R)], obj_v)
        for c in range(4):
            for v in range(CHUNK_R // 16):
                f = frac_v[c, pl.ds(v * 16, 16)]
                o = obj_v[pl.ds(v * 16, 16)]
                xi = (f * float(IMAGE_SIZE)).astype(jnp.int32)
                xi = jnp.minimum(jnp.maximum(xi, 0), IMAGE_SIZE)
                xi = jnp.where(o == 1, xi, ZROW)
                idx_v[c, pl.ds(v * 16, 16)] = xi
        descs = [
            pltpu.async_copy(table_hbm.at[idx_v.at[c]], rows_v.at[c], sem)
            for c in range(4)
        ]
        for d in descs:
            d.wait()
        for c in range(4):
            pltpu.sync_copy(rows_v.at[c], poe_hbm.at[c, pl.ds(off, CHUNK_R)])
        return carry

    lax.fori_loop(0, NCHUNK, chunk, 0)


def _sc_gather(frac_t, obj_flat, table):
    mesh = plsc.VectorSubcoreMesh(core_axis_name="c", subcore_axis_name="s")
    return functools.partial(
        pl.kernel,
        mesh=mesh,
        compiler_params=pltpu.CompilerParams(use_tc_tiling_on_sc=False),
        out_type=jax.ShapeDtypeStruct((4, ROWS, DG), jnp.float32),
        scratch_types=[
            pltpu.VMEM((4, CHUNK_R), jnp.float32),
            pltpu.VMEM((CHUNK_R,), jnp.int32),
            pltpu.VMEM((4, CHUNK_R), jnp.int32),
            pltpu.VMEM((4, CHUNK_R, DG), jnp.float32),
            pltpu.SemaphoreType.DMA,
        ],
    )(_sc_gather_body)(frac_t, obj_flat, table)


NB = 8
RB = NB * T


def _base_body(wt_ref, ww_ref, b_ref, out_ref):
    acc = (
        jnp.dot(wt_ref[...], ww_ref[...], preferred_element_type=jnp.float32)
        + b_ref[...]
    )
    for k in range(NB):
        out_ref[pl.ds(k * T, T), :] = acc


def _mm_body(poe_ref, wq_ref, base_ref, out_ref):
    acc = base_ref[...]
    for c in range(4):
        acc = acc + jnp.dot(
            poe_ref[c], wq_ref[c], preferred_element_type=jnp.float32
        )
    out_ref[...] = jnp.maximum(acc, 0.0)


def kernel(batch_fractional_bboxs, batch_obj_vecs, pos_table, word_table, W, b):
    frac_t = batch_fractional_bboxs.reshape(ROWS, 4).T
    obj_flat = batch_obj_vecs.reshape(ROWS)
    table = jnp.pad(pos_table, ((0, 3), (0, DG - D_POS)))
    Wq = jnp.pad(
        W[:BBOX_DIM].reshape(4, D_POS, OUT_DIM),
        ((0, 0), (0, DG - D_POS), (0, 0)),
    )
    Ww = W[BBOX_DIM:]

    base_rep = pl.pallas_call(
        _base_body,
        out_shape=jax.ShapeDtypeStruct((RB, OUT_DIM), jnp.float32),
    )(word_table, Ww, b.reshape(1, OUT_DIM))

    poe4 = _sc_gather(frac_t, obj_flat, table)

    return pl.pallas_call(
        _mm_body,
        grid=(ROWS // RB,),
        in_specs=[
            pl.BlockSpec((4, RB, DG), lambda i: (0, i, 0)),
            pl.BlockSpec((4, DG, OUT_DIM), lambda i: (0, 0, 0)),
            pl.BlockSpec((RB, OUT_DIM), lambda i: (0, 0)),
        ],
        out_specs=pl.BlockSpec((RB, OUT_DIM), lambda i: (i, 0)),
        out_shape=jax.ShapeDtypeStruct((ROWS, OUT_DIM), jnp.float32),
    )(poe4, Wq, base_rep)
